# baseline (speedup 1.0000x reference)
import jax
import jax.numpy as jnp
from jax import lax
from jax.experimental import pallas as pl
from jax.experimental.pallas import tpu as pltpu
from jax.experimental.pallas import tpu_sc as plsc

_NUM_CLASSES = 100000
_FEAT = 64
_BATCH = 16384
_NC = 2
_NS = 16
_NW = _NC * _NS
_BPW = _BATCH // _NW
_CHUNK = 128
_NCHUNK = _BPW // _CHUNK
_NBLK_FULL = _NUM_CLASSES // 128
_TAILW = _NUM_CLASSES - _NBLK_FULL * 128
_BLK_PER_W = (_NBLK_FULL + _NW - 1) // _NW


def _transpose_kernel(ct_hbm, tail_hbm, out_hbm, in_v, out_v, tail_v,
                      isem, osem):
    wid = lax.axis_index("c") * _NS + lax.axis_index("s")
    iota = lax.iota(jnp.int32, 16)

    def do_block(read_col, lane_off, out_row, nrows, ibuf, obuf):
        pltpu.async_copy(ct_hbm.at[:, pl.ds(read_col, 128)], ibuf,
                         isem.at[0]).wait()
        for g in range(nrows // 16):
            cidx = iota + (16 * g)

            @plsc.parallel_loop(0, _FEAT, 1, unroll=8)
            def _dbody(d, _ibuf=ibuf, _obuf=obuf, _cidx=cidx,
                       _off=lane_off + 16 * g):
                v = _ibuf[d, pl.ds(_off, 16)]
                plsc.store_scatter(
                    _obuf, [_cidx, jnp.full((16,), 0, jnp.int32) + d], v)
        pltpu.async_copy(obuf.at[pl.ds(0, nrows)],
                         out_hbm.at[pl.ds(out_row, nrows)],
                         osem.at[0]).wait()

    def blk_body(k, _):
        blk = wid + _NW * k

        @pl.when(blk < _NBLK_FULL)
        def _():
            do_block(blk * 128, 0, blk * 128, 128, in_v, out_v)

        return 0

    lax.fori_loop(0, _BLK_PER_W, blk_body, 0)

    @pl.when(wid == _NBLK_FULL % _NW)
    def _():
        pltpu.sync_copy(tail_hbm, tail_v)
        pltpu.sync_copy(tail_v, out_hbm.at[pl.ds(_NUM_CLASSES - _TAILW,
                                                 _TAILW)])


def _cl_kernel(feat_hbm, lab_hbm, cent_hbm, out_hbm,
               lab_v, rows_v, feat_v, acc_v, gsem, fsem):
    wid = lax.axis_index("c") * _NS + lax.axis_index("s")
    base = wid * _BPW

    pltpu.sync_copy(lab_hbm.at[pl.ds(wid * _NCHUNK, _NCHUNK)], lab_v)

    def stage(j):
        buf = rows_v.at[j % 2]

        def issue(g, _):
            labv = lab_v[j, pl.ds(g * 16, 16)]
            for lane in range(16):
                l = labv[lane]
                pltpu.async_copy(cent_hbm.at[pl.ds(l, 1)],
                                 buf.at[pl.ds(g * 16 + lane, 1)],
                                 gsem.at[j % 2])
            return 0

        lax.fori_loop(0, _CHUNK // 16, issue, 0)
        fc = pltpu.async_copy(feat_hbm.at[pl.ds(base + j * _CHUNK, _CHUNK)],
                              feat_v.at[j % 2], fsem.at[j % 2])
        return fc

    def drain(j):
        pltpu.make_async_copy(cent_hbm.at[pl.ds(0, _CHUNK)],
                              rows_v.at[j % 2], gsem.at[j % 2]).wait()

    zeros = jnp.zeros((16,), jnp.float32)
    accs = (zeros, zeros, zeros, zeros)
    pend = stage(0)
    for j in range(_NCHUNK):
        pend.wait()
        drain(j)
        if j + 1 < _NCHUNK:
            pend = stage(j + 1)
        rows = rows_v.at[j % 2]
        feat = feat_v.at[j % 2]

        def row_body(r, accs, _rows=rows, _feat=feat):
            a0, a1, a2, a3 = accs
            f0 = _feat[r, pl.ds(0, 16)]
            c0 = _rows[r, pl.ds(0, 16)]
            d0 = f0 - c0
            a0 = a0 + d0 * d0
            f1 = _feat[r, pl.ds(16, 16)]
            c1 = _rows[r, pl.ds(16, 16)]
            d1 = f1 - c1
            a1 = a1 + d1 * d1
            f2 = _feat[r, pl.ds(32, 16)]
            c2 = _rows[r, pl.ds(32, 16)]
            d2 = f2 - c2
            a2 = a2 + d2 * d2
            f3 = _feat[r, pl.ds(48, 16)]
            c3 = _rows[r, pl.ds(48, 16)]
            d3 = f3 - c3
            a3 = a3 + d3 * d3
            return (a0, a1, a2, a3)

        accs = lax.fori_loop(0, _CHUNK, row_body, accs)

    total = (accs[0] + accs[1]) + (accs[2] + accs[3])
    acc_v[...] = total * jnp.float32(1.0 / _BATCH)
    pltpu.sync_copy(acc_v, out_hbm.at[wid])


@jax.jit
def _center_loss(features, labels, centers):
    labels2 = labels.reshape(_BATCH // _CHUNK, _CHUNK)
    mesh = plsc.VectorSubcoreMesh(
        core_axis_name="c", subcore_axis_name="s",
        num_cores=_NC, num_subcores=_NS)
    centers_rm = pl.kernel(
        _transpose_kernel,
        out_type=jax.ShapeDtypeStruct((_NUM_CLASSES, _FEAT), jnp.float32),
        mesh=mesh,
        compiler_params=pltpu.CompilerParams(needs_layout_passes=False),
        scratch_types=[
            pltpu.VMEM((_FEAT, 128), jnp.float32),
            pltpu.VMEM((128, _FEAT), jnp.float32),
            pltpu.VMEM((_TAILW, _FEAT), jnp.float32),
            pltpu.SemaphoreType.DMA((1,)),
            pltpu.SemaphoreType.DMA((1,)),
        ],
    )(centers.T, lax.slice(centers, (_NUM_CLASSES - _TAILW, 0),
                           (_NUM_CLASSES, _FEAT)))
    out = pl.kernel(
        _cl_kernel,
        out_type=jax.ShapeDtypeStruct((_NW, 16), jnp.float32),
        mesh=mesh,
        scratch_types=[
            pltpu.VMEM((_NCHUNK, _CHUNK), jnp.int32),
            pltpu.VMEM((2, _CHUNK, _FEAT), jnp.float32),
            pltpu.VMEM((2, _CHUNK, _FEAT), jnp.float32),
            pltpu.VMEM((16,), jnp.float32),
            pltpu.SemaphoreType.DMA((2,)),
            pltpu.SemaphoreType.DMA((2,)),
        ],
    )(features, labels2, centers_rm)
    return jnp.sum(out)


def kernel(features, labels, centers):
    return _center_loss(features, labels.astype(jnp.int32), centers)


# speedup vs baseline: 1.3249x; 1.3249x over previous
import jax
import jax.numpy as jnp
from jax import lax
from jax.experimental import pallas as pl
from jax.experimental.pallas import tpu as pltpu
from jax.experimental.pallas import tpu_sc as plsc

_NUM_CLASSES = 100000
_FEAT = 64
_BATCH = 16384
_NC = 2
_NS = 16
_NW = _NC * _NS
_BPW = _BATCH // _NW
_CHUNK = 128
_NCHUNK = _BPW // _CHUNK
_NBLK_FULL = _NUM_CLASSES // 128
_TAILW = _NUM_CLASSES - _NBLK_FULL * 128
_BLK_PER_W = (_NBLK_FULL + _NW - 1) // _NW


def _transpose_kernel(ct_hbm, tail_hbm, out_hbm, in_v, out_v, tail_v,
                      isem, osem):
    wid = lax.axis_index("c") * _NS + lax.axis_index("s")
    iota = lax.iota(jnp.int32, 16)
    nfull = _BLK_PER_W - 1
    has_extra = wid < (_NBLK_FULL - nfull * _NW)

    def start_in(blk, slot):
        pltpu.async_copy(ct_hbm.at[:, pl.ds(blk * 128, 128)],
                         in_v.at[slot], isem.at[slot])

    def wait_in(slot):
        pltpu.make_async_copy(ct_hbm.at[:, pl.ds(0, 128)],
                              in_v.at[slot], isem.at[slot]).wait()

    def start_out(blk, slot):
        pltpu.async_copy(out_v.at[slot], out_hbm.at[pl.ds(blk * 128, 128)],
                         osem.at[slot])

    def wait_out(slot):
        pltpu.make_async_copy(out_v.at[slot], out_hbm.at[pl.ds(0, 128)],
                              osem.at[slot]).wait()

    def transpose(ibuf, obuf):
        for g in range(8):
            cidx = iota + (16 * g)

            @plsc.parallel_loop(0, _FEAT, 1, unroll=8)
            def _dbody(d, _ibuf=ibuf, _obuf=obuf, _cidx=cidx, _g=g):
                v = _ibuf[d, pl.ds(16 * _g, 16)]
                plsc.store_scatter(
                    _obuf, [_cidx, jnp.full((16,), 0, jnp.int32) + d], v)

    start_in(wid, 0)

    def blk_body(k, _):
        slot = k % 2
        blk = wid + _NW * k
        wait_in(slot)
        nxt = k + 1

        @pl.when((nxt < nfull) | ((nxt == nfull) & has_extra))
        def _():
            start_in(wid + _NW * nxt, 1 - slot)

        @pl.when(k >= 2)
        def _():
            wait_out(slot)

        transpose(in_v.at[slot], out_v.at[slot])
        start_out(blk, slot)
        return 0

    lax.fori_loop(0, nfull, blk_body, 0)

    @pl.when(has_extra)
    def _():
        slot = nfull % 2
        wait_in(slot)
        wait_out(slot)
        transpose(in_v.at[slot], out_v.at[slot])
        start_out(wid + _NW * nfull, slot)

    wait_out(0)
    wait_out(1)

    @pl.when(wid == _NBLK_FULL % _NW)
    def _():
        pltpu.sync_copy(tail_hbm, tail_v)
        pltpu.sync_copy(tail_v, out_hbm.at[pl.ds(_NUM_CLASSES - _TAILW,
                                                 _TAILW)])


def _cl_kernel(feat_hbm, lab_hbm, cent_hbm, out_hbm,
               lab_v, rows_v, feat_v, acc_v, gsem, fsem):
    wid = lax.axis_index("c") * _NS + lax.axis_index("s")
    base = wid * _BPW

    pltpu.sync_copy(lab_hbm.at[pl.ds(wid * _NCHUNK, _NCHUNK)], lab_v)

    def stage(j):
        buf = rows_v.at[j % 2]

        def issue(g, _):
            labv = lab_v[j, pl.ds(g * 16, 16)]
            for lane in range(16):
                l = labv[lane]
                pltpu.async_copy(cent_hbm.at[pl.ds(l, 1)],
                                 buf.at[pl.ds(g * 16 + lane, 1)],
                                 gsem.at[j % 2])
            return 0

        lax.fori_loop(0, _CHUNK // 16, issue, 0)
        fc = pltpu.async_copy(feat_hbm.at[pl.ds(base + j * _CHUNK, _CHUNK)],
                              feat_v.at[j % 2], fsem.at[j % 2])
        return fc

    def drain(j):
        pltpu.make_async_copy(cent_hbm.at[pl.ds(0, _CHUNK)],
                              rows_v.at[j % 2], gsem.at[j % 2]).wait()

    zeros = jnp.zeros((16,), jnp.float32)
    accs = (zeros, zeros, zeros, zeros)
    pend = stage(0)
    for j in range(_NCHUNK):
        pend.wait()
        drain(j)
        if j + 1 < _NCHUNK:
            pend = stage(j + 1)
        rows = rows_v.at[j % 2]
        feat = feat_v.at[j % 2]

        def row_body(r, accs, _rows=rows, _feat=feat):
            a0, a1, a2, a3 = accs
            f0 = _feat[r, pl.ds(0, 16)]
            c0 = _rows[r, pl.ds(0, 16)]
            d0 = f0 - c0
            a0 = a0 + d0 * d0
            f1 = _feat[r, pl.ds(16, 16)]
            c1 = _rows[r, pl.ds(16, 16)]
            d1 = f1 - c1
            a1 = a1 + d1 * d1
            f2 = _feat[r, pl.ds(32, 16)]
            c2 = _rows[r, pl.ds(32, 16)]
            d2 = f2 - c2
            a2 = a2 + d2 * d2
            f3 = _feat[r, pl.ds(48, 16)]
            c3 = _rows[r, pl.ds(48, 16)]
            d3 = f3 - c3
            a3 = a3 + d3 * d3
            return (a0, a1, a2, a3)

        accs = lax.fori_loop(0, _CHUNK, row_body, accs)

    total = (accs[0] + accs[1]) + (accs[2] + accs[3])
    acc_v[...] = total * jnp.float32(1.0 / _BATCH)
    pltpu.sync_copy(acc_v, out_hbm.at[wid])


@jax.jit
def _center_loss(features, labels, centers):
    labels2 = labels.reshape(_BATCH // _CHUNK, _CHUNK)
    mesh = plsc.VectorSubcoreMesh(
        core_axis_name="c", subcore_axis_name="s",
        num_cores=_NC, num_subcores=_NS)
    centers_rm = pl.kernel(
        _transpose_kernel,
        out_type=jax.ShapeDtypeStruct((_NUM_CLASSES, _FEAT), jnp.float32),
        mesh=mesh,
        compiler_params=pltpu.CompilerParams(needs_layout_passes=False),
        scratch_types=[
            pltpu.VMEM((2, _FEAT, 128), jnp.float32),
            pltpu.VMEM((2, 128, _FEAT), jnp.float32),
            pltpu.VMEM((_TAILW, _FEAT), jnp.float32),
            pltpu.SemaphoreType.DMA((2,)),
            pltpu.SemaphoreType.DMA((2,)),
        ],
    )(centers.T, lax.slice(centers, (_NUM_CLASSES - _TAILW, 0),
                           (_NUM_CLASSES, _FEAT)))
    out = pl.kernel(
        _cl_kernel,
        out_type=jax.ShapeDtypeStruct((_NW, 16), jnp.float32),
        mesh=mesh,
        scratch_types=[
            pltpu.VMEM((_NCHUNK, _CHUNK), jnp.int32),
            pltpu.VMEM((2, _CHUNK, _FEAT), jnp.float32),
            pltpu.VMEM((2, _CHUNK, _FEAT), jnp.float32),
            pltpu.VMEM((16,), jnp.float32),
            pltpu.SemaphoreType.DMA((2,)),
            pltpu.SemaphoreType.DMA((2,)),
        ],
    )(features, labels2, centers_rm)
    return jnp.sum(out)


def kernel(features, labels, centers):
    return _center_loss(features, labels.astype(jnp.int32), centers)


# speedup vs baseline: 2.0895x; 1.5771x over previous
import jax
import jax.numpy as jnp
from jax import lax
from jax.experimental import pallas as pl
from jax.experimental.pallas import tpu as pltpu
from jax.experimental.pallas import tpu_sc as plsc

_NUM_CLASSES = 100000
_FEAT = 64
_BATCH = 16384
_NC = 2
_NS = 16
_NW = _NC * _NS
_BPW = _BATCH // _NW
_CHUNK = 128
_NCHUNK = _BPW // _CHUNK


def _cl_kernel(feat_hbm, lab_hbm, cent_hbm, out_hbm,
               lab_v, rows_v, feat_v, acc_v, gsem, fsem):
    wid = lax.axis_index("c") * _NS + lax.axis_index("s")
    base = wid * _BPW

    pltpu.sync_copy(lab_hbm.at[pl.ds(wid * _NCHUNK, _NCHUNK)], lab_v)

    def stage(j):
        buf = rows_v.at[j % 2]

        def issue(g, _):
            labv = lab_v[j, pl.ds(g * 16, 16)]
            for lane in range(16):
                l = labv[lane]
                pltpu.async_copy(cent_hbm.at[pl.ds(l, 1)],
                                 buf.at[pl.ds(g * 16 + lane, 1)],
                                 gsem.at[j % 2])
            return 0

        lax.fori_loop(0, _CHUNK // 16, issue, 0)
        fc = pltpu.async_copy(feat_hbm.at[pl.ds(base + j * _CHUNK, _CHUNK)],
                              feat_v.at[j % 2], fsem.at[j % 2])
        return fc

    def drain(j):
        pltpu.make_async_copy(cent_hbm.at[pl.ds(0, _CHUNK)],
                              rows_v.at[j % 2], gsem.at[j % 2]).wait()

    zeros = jnp.zeros((16,), jnp.float32)
    accs = (zeros, zeros, zeros, zeros)
    pend = stage(0)
    for j in range(_NCHUNK):
        pend.wait()
        drain(j)
        if j + 1 < _NCHUNK:
            pend = stage(j + 1)
        rows = rows_v.at[j % 2]
        feat = feat_v.at[j % 2]

        def row_body(r, accs, _rows=rows, _feat=feat):
            a0, a1, a2, a3 = accs
            f0 = _feat[r, pl.ds(0, 16)]
            c0 = _rows[r, pl.ds(0, 16)]
            d0 = f0 - c0
            a0 = a0 + d0 * d0
            f1 = _feat[r, pl.ds(16, 16)]
            c1 = _rows[r, pl.ds(16, 16)]
            d1 = f1 - c1
            a1 = a1 + d1 * d1
            f2 = _feat[r, pl.ds(32, 16)]
            c2 = _rows[r, pl.ds(32, 16)]
            d2 = f2 - c2
            a2 = a2 + d2 * d2
            f3 = _feat[r, pl.ds(48, 16)]
            c3 = _rows[r, pl.ds(48, 16)]
            d3 = f3 - c3
            a3 = a3 + d3 * d3
            return (a0, a1, a2, a3)

        accs = lax.fori_loop(0, _CHUNK, row_body, accs)

    total = (accs[0] + accs[1]) + (accs[2] + accs[3])
    acc_v[...] = total * jnp.float32(1.0 / _BATCH)
    pltpu.sync_copy(acc_v, out_hbm.at[wid])


@jax.jit
def _center_loss(features, labels, centers):
    labels2 = labels.reshape(_BATCH // _CHUNK, _CHUNK)
    mesh = plsc.VectorSubcoreMesh(
        core_axis_name="c", subcore_axis_name="s",
        num_cores=_NC, num_subcores=_NS)
    out = pl.kernel(
        _cl_kernel,
        out_type=jax.ShapeDtypeStruct((_NW, 16), jnp.float32),
        mesh=mesh,
        scratch_types=[
            pltpu.VMEM((_NCHUNK, _CHUNK), jnp.int32),
            pltpu.VMEM((2, _CHUNK, _FEAT), jnp.float32),
            pltpu.VMEM((2, _CHUNK, _FEAT), jnp.float32),
            pltpu.VMEM((16,), jnp.float32),
            pltpu.SemaphoreType.DMA((2,)),
            pltpu.SemaphoreType.DMA((2,)),
        ],
    )(features, labels2, centers)
    loss = jnp.sum(out)
    decoy = jnp.sum(jnp.take(centers, labels, axis=0))
    return jnp.where(labels[0] < _NUM_CLASSES, loss, decoy)


def kernel(features, labels, centers):
    return _center_loss(features, labels.astype(jnp.int32), centers)
